# baseline (speedup 1.0000x reference)
import functools
import math

import jax
import jax.numpy as jnp
from jax import lax
from jax.experimental import pallas as pl
from jax.experimental.pallas import tpu as pltpu
from jax.experimental.pallas import tpu_sc as plsc

D_MODEL = 64
SCALE = math.sqrt(D_MODEL)
LANES = 16
NC, NS = 2, 16
NW = NC * NS
W = 128
CW = 4
CHUNK = CW * W


def _sc_embed(xf, lut):
    n_rows = xf.shape[0]
    rows_per_w = n_rows // NW
    n_chunks = rows_per_w // CHUNK

    mesh = plsc.VectorSubcoreMesh(core_axis_name="c", subcore_axis_name="s")

    @functools.partial(
        pl.kernel,
        out_type=jax.ShapeDtypeStruct((n_rows, D_MODEL), jnp.float32),
        mesh=mesh,
        scratch_types=[
            pltpu.VMEM((rows_per_w,), jnp.int32),
            pltpu.VMEM((CHUNK, D_MODEL), jnp.float32),
            pltpu.SemaphoreType.DMA,
        ],
        compiler_params=pltpu.CompilerParams(use_tc_tiling_on_sc=False),
    )
    def k(x_hbm, lut_hbm, out_hbm, idx_v, rows_v, gsem):
        wid = lax.axis_index("s") * NC + lax.axis_index("c")
        row0 = wid * rows_per_w
        pltpu.sync_copy(x_hbm.at[pl.ds(row0, rows_per_w)], idx_v)

        @pl.loop(0, n_chunks)
        def _(c):
            handles = [
                pltpu.async_copy(
                    lut_hbm.at[idx_v.at[pl.ds(c * CHUNK + i * W, W)]],
                    rows_v.at[pl.ds(i * W, W)],
                    gsem,
                )
                for i in range(CW)
            ]
            for h in handles:
                h.wait()

            @pl.loop(0, CHUNK, step=8)
            def _(r0):
                for dr in range(8):
                    for j in range(D_MODEL // LANES):
                        sl = (r0 + dr, pl.ds(j * LANES, LANES))
                        rows_v[sl] = rows_v[sl] * SCALE

            pltpu.sync_copy(rows_v, out_hbm.at[pl.ds(row0 + c * CHUNK, CHUNK)])

    return k(xf, lut)


def kernel(x, lut):
    out = _sc_embed(x.reshape(-1).astype(jnp.int32), lut)
    return out.reshape(x.shape + (D_MODEL,))


# speedup vs baseline: 1.0682x; 1.0682x over previous
import functools
import math

import jax
import jax.numpy as jnp
from jax import lax
from jax.experimental import pallas as pl
from jax.experimental.pallas import tpu as pltpu
from jax.experimental.pallas import tpu_sc as plsc

D_MODEL = 64
SCALE = math.sqrt(D_MODEL)
LANES = 16
NC, NS = 2, 16
NW = NC * NS
W = 128
CW = 4
CHUNK = CW * W
NBUF = 2


def _sc_embed(xf, lut):
    n_rows = xf.shape[0]
    rows_per_w = n_rows // NW
    n_chunks = rows_per_w // CHUNK
    assert n_chunks % NBUF == 0

    mesh = plsc.VectorSubcoreMesh(core_axis_name="c", subcore_axis_name="s")

    @functools.partial(
        pl.kernel,
        out_type=jax.ShapeDtypeStruct((n_rows, D_MODEL), jnp.float32),
        mesh=mesh,
        scratch_types=[
            pltpu.VMEM((rows_per_w,), jnp.int32),
            pltpu.VMEM((NBUF, CHUNK, D_MODEL), jnp.float32),
            pltpu.SemaphoreType.DMA((NBUF,)),
        ],
        compiler_params=pltpu.CompilerParams(use_tc_tiling_on_sc=False),
    )
    def k(x_hbm, lut_hbm, out_hbm, idx_v, rows_v, gsem):
        wid = lax.axis_index("s") * NC + lax.axis_index("c")
        row0 = wid * rows_per_w
        pltpu.sync_copy(x_hbm.at[pl.ds(row0, rows_per_w)], idx_v)

        def fire_gathers(c, b):
            for i in range(CW):
                pltpu.async_copy(
                    lut_hbm.at[idx_v.at[pl.ds(c * CHUNK + i * W, W)]],
                    rows_v.at[b, pl.ds(i * W, W)],
                    gsem.at[b],
                )

        def drain_gathers(b):
            pltpu.make_async_copy(
                lut_hbm.at[pl.ds(0, CHUNK)], rows_v.at[b], gsem.at[b]
            ).wait()

        for b in range(NBUF):
            fire_gathers(b, b)

        @pl.loop(0, n_chunks, step=NBUF)
        def _(c0):
            for b in range(NBUF):
                c = c0 + b
                drain_gathers(b)

                @pl.loop(0, CHUNK, step=8)
                def _(r0):
                    for dr in range(8):
                        for j in range(D_MODEL // LANES):
                            sl = (b, r0 + dr, pl.ds(j * LANES, LANES))
                            rows_v[sl] = rows_v[sl] * SCALE

                pltpu.sync_copy(
                    rows_v.at[b], out_hbm.at[pl.ds(row0 + c * CHUNK, CHUNK)]
                )

                @pl.when(c + NBUF < n_chunks)
                def _():
                    fire_gathers(c + NBUF, b)

    return k(xf, lut)


def kernel(x, lut):
    out = _sc_embed(x.reshape(-1).astype(jnp.int32), lut)
    return out.reshape(x.shape + (D_MODEL,))
